# Initial kernel scaffold; baseline (speedup 1.0000x reference)
import jax
import jax.numpy as jnp
from jax.experimental import pallas as pl


def kernel(x, Wg, bg, W1, b1, W2, b2):
    raise NotImplementedError("write your pallas kernel here")


# speedup vs baseline: 1.7670x; 1.7670x over previous
import jax
import jax.numpy as jnp
from jax.experimental import pallas as pl
from jax.experimental.pallas import tpu as pltpu

D_MODEL = 768
HIDDEN = 768
OUT_D = 768
E = 8
TOPK = 2
T = 2048


def _moe_kernel(x_ref, Wg_ref, bg_ref, W1_ref, b1_ref, W2_ref, b2_ref,
                out_ref, combine_ref):
    e = pl.program_id(0)

    @pl.when(e == 0)
    def _gating():
        x = x_ref[...]
        logits = jnp.dot(x, Wg_ref[...]) + bg_ref[...]
        probs = jax.nn.softmax(logits, axis=-1)
        col = jax.lax.broadcasted_iota(jnp.int32, probs.shape, 1)
        m1 = jnp.max(probs, axis=-1, keepdims=True)
        idx1 = jnp.min(jnp.where(probs == m1, col, E), axis=-1, keepdims=True)
        mask1 = col == idx1
        probsm = jnp.where(mask1, -jnp.inf, probs)
        m2 = jnp.max(probsm, axis=-1, keepdims=True)
        idx2 = jnp.min(jnp.where(probsm == m2, col, E), axis=-1, keepdims=True)
        mask2 = col == idx2
        denom = m1 + m2 + 1e-9
        combine_ref[...] = jnp.where(mask1 | mask2, probs, 0.0) / denom

    x = x_ref[...]
    h = jax.nn.relu(jnp.dot(x, W1_ref[0]) + b1_ref[0])
    y = jnp.dot(h, W2_ref[0]) + b2_ref[0]
    col = jax.lax.broadcasted_iota(jnp.int32, (T, E), 1)
    c = jnp.sum(jnp.where(col == e, combine_ref[...], 0.0), axis=-1,
                keepdims=True)

    @pl.when(e == 0)
    def _init():
        out_ref[...] = c * y

    @pl.when(e != 0)
    def _acc():
        out_ref[...] += c * y


def kernel(x, Wg, bg, W1, b1, W2, b2):
    bg2 = bg.reshape(1, E)
    b1r = b1.reshape(E, 1, HIDDEN)
    b2r = b2.reshape(E, 1, OUT_D)
    return pl.pallas_call(
        _moe_kernel,
        grid=(E,),
        in_specs=[
            pl.BlockSpec((T, D_MODEL), lambda e: (0, 0)),
            pl.BlockSpec((D_MODEL, E), lambda e: (0, 0)),
            pl.BlockSpec((1, E), lambda e: (0, 0)),
            pl.BlockSpec((1, D_MODEL, HIDDEN), lambda e: (e, 0, 0)),
            pl.BlockSpec((1, 1, HIDDEN), lambda e: (e, 0, 0)),
            pl.BlockSpec((1, HIDDEN, OUT_D), lambda e: (e, 0, 0)),
            pl.BlockSpec((1, 1, OUT_D), lambda e: (e, 0, 0)),
        ],
        out_specs=pl.BlockSpec((T, OUT_D), lambda e: (0, 0)),
        out_shape=jax.ShapeDtypeStruct((T, OUT_D), x.dtype),
        scratch_shapes=[pltpu.VMEM((T, E), jnp.float32)],
        compiler_params=pltpu.CompilerParams(
            dimension_semantics=("arbitrary",),
        ),
    )(x, Wg, bg2, W1, b1r, W2, b2r)


# speedup vs baseline: 1.7739x; 1.0039x over previous
import jax
import jax.numpy as jnp
from jax.experimental import pallas as pl
from jax.experimental.pallas import tpu as pltpu

D_MODEL = 768
HIDDEN = 768
OUT_D = 768
E = 8
TOPK = 2
T = 2048


def _moe_kernel(x_ref, Wg_ref, bg_ref, W1_ref, b1_ref, W2_ref, b2_ref,
                out_ref, combine_ref):
    e = pl.program_id(0)

    @pl.when(e == 0)
    def _gating():
        logits = jax.lax.dot_general(
            Wg_ref[...], x_ref[...], (((0,), (1,)), ((), ())),
            preferred_element_type=jnp.float32) + bg_ref[...]
        m = jnp.max(logits, axis=0, keepdims=True)
        ex = jnp.exp(logits - m)
        probs = ex / jnp.sum(ex, axis=0, keepdims=True)
        row = jax.lax.broadcasted_iota(jnp.int32, probs.shape, 0)
        m1 = jnp.max(probs, axis=0, keepdims=True)
        idx1 = jnp.min(jnp.where(probs == m1, row, E), axis=0, keepdims=True)
        mask1 = row == idx1
        probsm = jnp.where(mask1, -jnp.inf, probs)
        m2 = jnp.max(probsm, axis=0, keepdims=True)
        idx2 = jnp.min(jnp.where(probsm == m2, row, E), axis=0, keepdims=True)
        mask2 = row == idx2
        denom = m1 + m2 + 1e-9
        combine_t = jnp.where(mask1 | mask2, probs, 0.0) / denom
        combine_ref[...] = combine_t.T

    x = x_ref[...]
    h = jax.nn.relu(jnp.dot(x, W1_ref[0],
                            preferred_element_type=jnp.float32) + b1_ref[0])
    y = jnp.dot(h, W2_ref[0],
                preferred_element_type=jnp.float32) + b2_ref[0]
    col = jax.lax.broadcasted_iota(jnp.int32, (T, E), 1)
    c = jnp.sum(jnp.where(col == e, combine_ref[...], 0.0), axis=-1,
                keepdims=True)

    @pl.when(e == 0)
    def _init():
        out_ref[...] = c * y

    @pl.when(e != 0)
    def _acc():
        out_ref[...] += c * y


def kernel(x, Wg, bg, W1, b1, W2, b2):
    bg2 = bg.reshape(E, 1)
    b1r = b1.reshape(E, 1, HIDDEN)
    b2r = b2.reshape(E, 1, OUT_D)
    return pl.pallas_call(
        _moe_kernel,
        grid=(E,),
        in_specs=[
            pl.BlockSpec((T, D_MODEL), lambda e: (0, 0)),
            pl.BlockSpec((D_MODEL, E), lambda e: (0, 0)),
            pl.BlockSpec((E, 1), lambda e: (0, 0)),
            pl.BlockSpec((1, D_MODEL, HIDDEN), lambda e: (e, 0, 0)),
            pl.BlockSpec((1, 1, HIDDEN), lambda e: (e, 0, 0)),
            pl.BlockSpec((1, HIDDEN, OUT_D), lambda e: (e, 0, 0)),
            pl.BlockSpec((1, 1, OUT_D), lambda e: (e, 0, 0)),
        ],
        out_specs=pl.BlockSpec((T, OUT_D), lambda e: (0, 0)),
        out_shape=jax.ShapeDtypeStruct((T, OUT_D), x.dtype),
        scratch_shapes=[pltpu.VMEM((T, E), jnp.float32)],
        compiler_params=pltpu.CompilerParams(
            dimension_semantics=("arbitrary",),
        ),
    )(x, Wg, bg2, W1, b1r, W2, b2r)


# speedup vs baseline: 1.8968x; 1.0693x over previous
import jax
import jax.numpy as jnp
from jax.experimental import pallas as pl
from jax.experimental.pallas import tpu as pltpu

D_MODEL = 768
HIDDEN = 768
OUT_D = 768
E = 8
TOPK = 2
T = 2048


def _moe_kernel(x_ref, Wg_ref, bg_ref, b2all_ref, W1_ref, b1_ref, W2_ref,
                out_ref, cvec_ref):
    e = pl.program_id(0)

    @pl.when(e == 0)
    def _gating():
        logits = jax.lax.dot_general(
            Wg_ref[...], x_ref[...], (((0,), (1,)), ((), ())),
            preferred_element_type=jnp.float32) + bg_ref[...]
        m = jnp.max(logits, axis=0, keepdims=True)
        ex = jnp.exp(logits - m)
        probs = ex / jnp.sum(ex, axis=0, keepdims=True)
        row = jax.lax.broadcasted_iota(jnp.int32, probs.shape, 0)
        m1 = jnp.max(probs, axis=0, keepdims=True)
        idx1 = jnp.min(jnp.where(probs == m1, row, E), axis=0, keepdims=True)
        mask1 = row == idx1
        probsm = jnp.where(mask1, -jnp.inf, probs)
        m2 = jnp.max(probsm, axis=0, keepdims=True)
        idx2 = jnp.min(jnp.where(probsm == m2, row, E), axis=0, keepdims=True)
        mask2 = row == idx2
        denom = m1 + m2 + 1e-9
        combine_t = jnp.where(mask1 | mask2, probs, 0.0) / denom
        cvec_ref[...] = combine_t[:, :, None]
        out_ref[...] = jax.lax.dot_general(
            combine_t, b2all_ref[...], (((0,), (0,)), ((), ())),
            preferred_element_type=jnp.float32)

    h = jax.nn.relu(jnp.dot(x_ref[...], W1_ref[0],
                            preferred_element_type=jnp.float32) + b1_ref[0])
    y = jnp.dot(h, W2_ref[0], preferred_element_type=jnp.float32)
    out_ref[...] += cvec_ref[e] * y


def kernel(x, Wg, bg, W1, b1, W2, b2):
    bg2 = bg.reshape(E, 1)
    b1r = b1.reshape(E, 1, HIDDEN)
    return pl.pallas_call(
        _moe_kernel,
        grid=(E,),
        in_specs=[
            pl.BlockSpec((T, D_MODEL), lambda e: (0, 0)),
            pl.BlockSpec((D_MODEL, E), lambda e: (0, 0)),
            pl.BlockSpec((E, 1), lambda e: (0, 0)),
            pl.BlockSpec((E, OUT_D), lambda e: (0, 0)),
            pl.BlockSpec((1, D_MODEL, HIDDEN), lambda e: (e, 0, 0)),
            pl.BlockSpec((1, 1, HIDDEN), lambda e: (e, 0, 0)),
            pl.BlockSpec((1, HIDDEN, OUT_D), lambda e: (e, 0, 0)),
        ],
        out_specs=pl.BlockSpec((T, OUT_D), lambda e: (0, 0)),
        out_shape=jax.ShapeDtypeStruct((T, OUT_D), x.dtype),
        scratch_shapes=[pltpu.VMEM((E, T, 1), jnp.float32)],
        compiler_params=pltpu.CompilerParams(
            dimension_semantics=("arbitrary",),
        ),
    )(x, Wg, bg2, b2, W1, b1r, W2)
